# baseline (device time: 82868 ns/iter reference)
import jax
import jax.numpy as jnp
from jax import lax
from jax.experimental import pallas as pl
from jax.experimental.pallas import tpu as pltpu

N_LINE = 4
R = 128
OFFSETS = (1, -1, 2, -2, 3, -3)
D2I = {1: 0, -1: 1, 2: 2, -2: 3, 3: 4, -3: 5}


def kernel(partial, resid, gamma):
    _, M, D = partial.shape
    HC = D // 2

    def body(partial_ref, resid_ref, gamma_ref, out_ref,
             gathA, gathB, pstage, rstage, psend, precv,
             send_x, recv_x, dma_p, dma_r, ag_send, ag_recv):
        my_x = lax.axis_index("x")
        my_y = lax.axis_index("y")
        my_z = lax.axis_index("z")
        xpeer = (1 - my_x, my_y, my_z)

        def z_peer(d):
            return (my_x, my_y, my_z + d)

        def y_peer(d):
            return (my_x, my_y + d, my_z)

        barrier_sem = pltpu.get_barrier_semaphore()
        pl.semaphore_signal(barrier_sem, inc=1, device_id=xpeer,
                            device_id_type=pl.DeviceIdType.MESH)
        for peer_fn, pos in ((y_peer, my_y), (z_peer, my_z)):
            for d in OFFSETS:
                @pl.when((pos + d >= 0) & (pos + d <= N_LINE - 1))
                def _(peer_fn=peer_fn, d=d):
                    pl.semaphore_signal(
                        barrier_sem, inc=1, device_id=peer_fn(d),
                        device_id_type=pl.DeviceIdType.MESH)
        pl.semaphore_wait(barrier_sem, 7)

        c_me = N_LINE * my_y + my_z
        row0 = c_me * R
        cp = pltpu.make_async_copy(
            partial_ref.at[0, pl.ds(row0, R)], pstage, dma_p)
        cp.start()
        cr = pltpu.make_async_copy(
            resid_ref.at[pl.ds(row0, R)], rstage, dma_r)
        cr.start()
        cp.wait()
        psend[...] = pstage[...].astype(jnp.bfloat16)
        rx = pltpu.make_async_remote_copy(
            src_ref=psend, dst_ref=precv, send_sem=send_x, recv_sem=recv_x,
            device_id=xpeer, device_id_type=pl.DeviceIdType.MESH)
        rx.start()
        rx.wait()
        cr.wait()

        y = (psend[...].astype(jnp.float32)
             + precv[...].astype(jnp.float32)
             + rstage[...])
        rms = jnp.sqrt(jnp.mean(y * y, axis=-1, keepdims=True) + 1e-6)
        o = y / rms * gamma_ref[...][None, :]
        gathA[my_y, my_z] = o[:, :HC].astype(jnp.bfloat16)
        gathB[my_z, my_y] = o[:, HC:].astype(jnp.bfloat16)

        def unit_A_z(i):
            return gathA.at[my_y, i]

        def unit_B_y(i):
            return gathB.at[my_z, i]

        def unit_A_y(i):
            return gathA.at[i]

        def unit_B_z(i):
            return gathB.at[i]

        def descr(phase, d, src_i, dst_i, unit, peer_fn):
            return pltpu.make_async_remote_copy(
                src_ref=unit(src_i), dst_ref=unit(dst_i),
                send_sem=ag_send.at[phase, D2I[d]],
                recv_sem=ag_recv.at[phase, D2I[d]],
                device_id=peer_fn(d),
                device_id_type=pl.DeviceIdType.MESH)

        def ag_start(phase, pos, unit, peer_fn):
            for d in OFFSETS:
                @pl.when((pos + d >= 0) & (pos + d <= N_LINE - 1))
                def _(d=d):
                    descr(phase, d, pos, pos, unit, peer_fn).start()

        def ag_wait_recv(phase, pos, unit, peer_fn):
            for d in OFFSETS:
                @pl.when((pos - d >= 0) & (pos - d <= N_LINE - 1))
                def _(d=d):
                    descr(phase, d, pos - d, pos - d, unit,
                          peer_fn).wait_recv()

        def ag_wait_send(phase, pos, unit, peer_fn):
            for d in OFFSETS:
                @pl.when((pos + d >= 0) & (pos + d <= N_LINE - 1))
                def _(d=d):
                    descr(phase, d, pos, pos, unit, peer_fn).wait_send()

        ag_start(0, my_z, unit_A_z, z_peer)
        ag_start(1, my_y, unit_B_y, y_peer)
        ag_wait_recv(0, my_z, unit_A_z, z_peer)
        ag_wait_recv(1, my_y, unit_B_y, y_peer)
        ag_wait_send(0, my_z, unit_A_z, z_peer)
        ag_wait_send(1, my_y, unit_B_y, y_peer)

        ag_start(2, my_y, unit_A_y, y_peer)
        ag_start(3, my_z, unit_B_z, z_peer)
        ag_wait_recv(2, my_y, unit_A_y, y_peer)
        ag_wait_recv(3, my_z, unit_B_z, z_peer)
        ag_wait_send(2, my_y, unit_A_y, y_peer)
        ag_wait_send(3, my_z, unit_B_z, z_peer)

        for yy in range(N_LINE):
            for zz in range(N_LINE):
                c = N_LINE * yy + zz
                rows = pl.ds(c * R, R)
                out_ref[rows, 0:HC] = gathA[yy, zz].astype(jnp.float32)
                out_ref[rows, HC:D] = gathB[zz, yy].astype(jnp.float32)

    return pl.pallas_call(
        body,
        out_shape=jax.ShapeDtypeStruct((M, D), jnp.float32),
        in_specs=[
            pl.BlockSpec(memory_space=pltpu.MemorySpace.HBM),
            pl.BlockSpec(memory_space=pltpu.MemorySpace.HBM),
            pl.BlockSpec(memory_space=pltpu.MemorySpace.VMEM),
        ],
        out_specs=pl.BlockSpec(memory_space=pltpu.MemorySpace.VMEM),
        scratch_shapes=[
            pltpu.VMEM((N_LINE, N_LINE, R, HC), jnp.bfloat16),
            pltpu.VMEM((N_LINE, N_LINE, R, HC), jnp.bfloat16),
            pltpu.VMEM((R, D), jnp.float32),
            pltpu.VMEM((R, D), jnp.float32),
            pltpu.VMEM((R, D), jnp.bfloat16),
            pltpu.VMEM((R, D), jnp.bfloat16),
            pltpu.SemaphoreType.DMA,
            pltpu.SemaphoreType.DMA,
            pltpu.SemaphoreType.DMA,
            pltpu.SemaphoreType.DMA,
            pltpu.SemaphoreType.DMA((4, 6)),
            pltpu.SemaphoreType.DMA((4, 6)),
        ],
        compiler_params=pltpu.CompilerParams(collective_id=0),
    )(partial, resid, gamma)


# device time: 71244 ns/iter; 1.1632x vs baseline; 1.1632x over previous
import jax
import jax.numpy as jnp
from jax import lax
from jax.experimental import pallas as pl
from jax.experimental.pallas import tpu as pltpu

N_LINE = 4
R = 128


def kernel(partial, resid, gamma):
    _, M, D = partial.shape
    HC = D // 2

    def body(partial_ref, resid_ref, gamma_ref, out_ref,
             gathA, gathB, pstage, rstage, psend, precv,
             send_x, recv_x, dma_p, dma_r, ag_send, ag_recv):
        my_x = lax.axis_index("x")
        my_y = lax.axis_index("y")
        my_z = lax.axis_index("z")
        xpeer = (1 - my_x, my_y, my_z)

        barrier_sem = pltpu.get_barrier_semaphore()
        pl.semaphore_signal(barrier_sem, inc=1, device_id=xpeer,
                            device_id_type=pl.DeviceIdType.MESH)

        @pl.when(my_y > 0)
        def _():
            pl.semaphore_signal(barrier_sem, inc=1,
                                device_id=(my_x, my_y - 1, my_z),
                                device_id_type=pl.DeviceIdType.MESH)

        @pl.when(my_y < N_LINE - 1)
        def _():
            pl.semaphore_signal(barrier_sem, inc=1,
                                device_id=(my_x, my_y + 1, my_z),
                                device_id_type=pl.DeviceIdType.MESH)

        @pl.when(my_z > 0)
        def _():
            pl.semaphore_signal(barrier_sem, inc=1,
                                device_id=(my_x, my_y, my_z - 1),
                                device_id_type=pl.DeviceIdType.MESH)

        @pl.when(my_z < N_LINE - 1)
        def _():
            pl.semaphore_signal(barrier_sem, inc=1,
                                device_id=(my_x, my_y, my_z + 1),
                                device_id_type=pl.DeviceIdType.MESH)

        c_me = N_LINE * my_y + my_z
        row0 = c_me * R
        cp = pltpu.make_async_copy(
            partial_ref.at[0, pl.ds(row0, R)], pstage, dma_p)
        cp.start()
        cr = pltpu.make_async_copy(
            resid_ref.at[pl.ds(row0, R)], rstage, dma_r)
        cr.start()
        cp.wait()
        psend[...] = pstage[...].astype(jnp.bfloat16)

        n_nbrs = (1
                  + (my_y > 0).astype(jnp.int32)
                  + (my_y < N_LINE - 1).astype(jnp.int32)
                  + (my_z > 0).astype(jnp.int32)
                  + (my_z < N_LINE - 1).astype(jnp.int32))
        pl.semaphore_wait(barrier_sem, n_nbrs)

        rx = pltpu.make_async_remote_copy(
            src_ref=psend, dst_ref=precv, send_sem=send_x, recv_sem=recv_x,
            device_id=xpeer, device_id_type=pl.DeviceIdType.MESH)
        rx.start()
        rx.wait()
        cr.wait()

        y = (psend[...].astype(jnp.float32)
             + precv[...].astype(jnp.float32)
             + rstage[...])
        rms = jnp.sqrt(jnp.mean(y * y, axis=-1, keepdims=True) + 1e-6)
        o = (y / rms * gamma_ref[...][None, :]).astype(jnp.bfloat16)
        gathA[pl.ds(c_me * R, R), :] = o[:, :HC]
        gathB[pl.ds((N_LINE * my_z + my_y) * R, R), :] = o[:, HC:]

        def z_peer(d):
            return (my_x, my_y, my_z + d)

        def y_peer(d):
            return (my_x, my_y + d, my_z)

        def unit_A_z(i):
            return gathA.at[pl.ds((N_LINE * my_y + i) * R, R)]

        def unit_B_y(i):
            return gathB.at[pl.ds((N_LINE * my_z + i) * R, R)]

        def unit_A_y(i):
            return gathA.at[pl.ds(i * N_LINE * R, N_LINE * R)]

        def unit_B_z(i):
            return gathB.at[pl.ds(i * N_LINE * R, N_LINE * R)]

        def step_send(phase, s, pos, unit, peer_fn):
            @pl.when((pos >= s) & (pos < N_LINE - 1))
            def _():
                r = pltpu.make_async_remote_copy(
                    src_ref=unit(pos - s), dst_ref=unit(pos - s),
                    send_sem=ag_send.at[phase, s, 0],
                    recv_sem=ag_recv.at[phase, s, 0],
                    device_id=peer_fn(1),
                    device_id_type=pl.DeviceIdType.MESH)
                r.start()

            @pl.when((pos > 0) & (pos + s <= N_LINE - 1))
            def _():
                r = pltpu.make_async_remote_copy(
                    src_ref=unit(pos + s), dst_ref=unit(pos + s),
                    send_sem=ag_send.at[phase, s, 1],
                    recv_sem=ag_recv.at[phase, s, 1],
                    device_id=peer_fn(-1),
                    device_id_type=pl.DeviceIdType.MESH)
                r.start()

        def step_wait(phase, s, pos, unit, peer_fn):
            @pl.when(pos >= s + 1)
            def _():
                r = pltpu.make_async_remote_copy(
                    src_ref=unit(pos - 1 - s), dst_ref=unit(pos - 1 - s),
                    send_sem=ag_send.at[phase, s, 0],
                    recv_sem=ag_recv.at[phase, s, 0],
                    device_id=peer_fn(-1),
                    device_id_type=pl.DeviceIdType.MESH)
                r.wait_recv()

            @pl.when(pos + 1 + s <= N_LINE - 1)
            def _():
                r = pltpu.make_async_remote_copy(
                    src_ref=unit(pos + 1 + s), dst_ref=unit(pos + 1 + s),
                    send_sem=ag_send.at[phase, s, 1],
                    recv_sem=ag_recv.at[phase, s, 1],
                    device_id=peer_fn(1),
                    device_id_type=pl.DeviceIdType.MESH)
                r.wait_recv()

            @pl.when((pos >= s) & (pos < N_LINE - 1))
            def _():
                r = pltpu.make_async_remote_copy(
                    src_ref=unit(pos - s), dst_ref=unit(pos - s),
                    send_sem=ag_send.at[phase, s, 0],
                    recv_sem=ag_recv.at[phase, s, 0],
                    device_id=peer_fn(1),
                    device_id_type=pl.DeviceIdType.MESH)
                r.wait_send()

            @pl.when((pos > 0) & (pos + s <= N_LINE - 1))
            def _():
                r = pltpu.make_async_remote_copy(
                    src_ref=unit(pos + s), dst_ref=unit(pos + s),
                    send_sem=ag_send.at[phase, s, 1],
                    recv_sem=ag_recv.at[phase, s, 1],
                    device_id=peer_fn(-1),
                    device_id_type=pl.DeviceIdType.MESH)
                r.wait_send()

        for s in range(N_LINE - 1):
            step_send(0, s, my_z, unit_A_z, z_peer)
            step_send(1, s, my_y, unit_B_y, y_peer)
            step_wait(0, s, my_z, unit_A_z, z_peer)
            step_wait(1, s, my_y, unit_B_y, y_peer)

        for s in range(N_LINE - 1):
            step_send(2, s, my_y, unit_A_y, y_peer)
            step_send(3, s, my_z, unit_B_z, z_peer)
            step_wait(2, s, my_y, unit_A_y, y_peer)
            step_wait(3, s, my_z, unit_B_z, z_peer)

        out_ref[:, 0:HC] = gathA[:, :]
        for yy in range(N_LINE):
            for zz in range(N_LINE):
                rows = pl.ds((N_LINE * yy + zz) * R, R)
                out_ref[rows, HC:D] = gathB[pl.ds((N_LINE * zz + yy) * R, R), :]

    return pl.pallas_call(
        body,
        out_shape=jax.ShapeDtypeStruct((M, D), jnp.bfloat16),
        in_specs=[
            pl.BlockSpec(memory_space=pltpu.MemorySpace.HBM),
            pl.BlockSpec(memory_space=pltpu.MemorySpace.HBM),
            pl.BlockSpec(memory_space=pltpu.MemorySpace.VMEM),
        ],
        out_specs=pl.BlockSpec(memory_space=pltpu.MemorySpace.VMEM),
        scratch_shapes=[
            pltpu.VMEM((M, HC), jnp.bfloat16),
            pltpu.VMEM((M, HC), jnp.bfloat16),
            pltpu.VMEM((R, D), jnp.float32),
            pltpu.VMEM((R, D), jnp.float32),
            pltpu.VMEM((R, D), jnp.bfloat16),
            pltpu.VMEM((R, D), jnp.bfloat16),
            pltpu.SemaphoreType.DMA,
            pltpu.SemaphoreType.DMA,
            pltpu.SemaphoreType.DMA,
            pltpu.SemaphoreType.DMA,
            pltpu.SemaphoreType.DMA((4, N_LINE - 1, 2)),
            pltpu.SemaphoreType.DMA((4, N_LINE - 1, 2)),
        ],
        compiler_params=pltpu.CompilerParams(collective_id=0),
    )(partial, resid, gamma)


# device time: 71221 ns/iter; 1.1635x vs baseline; 1.0003x over previous
import jax
import jax.numpy as jnp
from jax import lax
from jax.experimental import pallas as pl
from jax.experimental.pallas import tpu as pltpu

N_LINE = 4
R = 128


def kernel(partial, resid, gamma):
    _, M, D = partial.shape
    HC = D // 2

    def body(partial_ref, resid_ref, gamma_ref, out_ref,
             gathA, gathB, pstage, rstage, psend, precv,
             send_x, recv_x, dma_p, dma_r, ag_send, ag_recv):
        my_x = lax.axis_index("x")
        my_y = lax.axis_index("y")
        my_z = lax.axis_index("z")
        xpeer = (1 - my_x, my_y, my_z)

        barrier_sem = pltpu.get_barrier_semaphore()
        pl.semaphore_signal(barrier_sem, inc=1, device_id=xpeer,
                            device_id_type=pl.DeviceIdType.MESH)

        @pl.when(my_y > 0)
        def _():
            pl.semaphore_signal(barrier_sem, inc=1,
                                device_id=(my_x, my_y - 1, my_z),
                                device_id_type=pl.DeviceIdType.MESH)

        @pl.when(my_y < N_LINE - 1)
        def _():
            pl.semaphore_signal(barrier_sem, inc=1,
                                device_id=(my_x, my_y + 1, my_z),
                                device_id_type=pl.DeviceIdType.MESH)

        @pl.when(my_z > 0)
        def _():
            pl.semaphore_signal(barrier_sem, inc=1,
                                device_id=(my_x, my_y, my_z - 1),
                                device_id_type=pl.DeviceIdType.MESH)

        @pl.when(my_z < N_LINE - 1)
        def _():
            pl.semaphore_signal(barrier_sem, inc=1,
                                device_id=(my_x, my_y, my_z + 1),
                                device_id_type=pl.DeviceIdType.MESH)

        c_me = N_LINE * my_y + my_z
        row0 = c_me * R
        cp = pltpu.make_async_copy(
            partial_ref.at[0, pl.ds(row0, R)], pstage, dma_p)
        cp.start()
        cr = pltpu.make_async_copy(
            resid_ref.at[pl.ds(row0, R)], rstage, dma_r)
        cr.start()
        cp.wait()
        psend[...] = pstage[...].astype(jnp.bfloat16)

        _nsc_stage = jax.named_scope("stage"); _nsc_stage.__enter__()
        n_nbrs = (1
                  + (my_y > 0).astype(jnp.int32)
                  + (my_y < N_LINE - 1).astype(jnp.int32)
                  + (my_z > 0).astype(jnp.int32)
                  + (my_z < N_LINE - 1).astype(jnp.int32))
        _nsc_stage.__exit__(None, None, None)
        with jax.named_scope("barrier_wait"):
            pl.semaphore_wait(barrier_sem, n_nbrs)

        with jax.named_scope("xchg"):
            rx = pltpu.make_async_remote_copy(
                src_ref=psend, dst_ref=precv, send_sem=send_x, recv_sem=recv_x,
                device_id=xpeer, device_id_type=pl.DeviceIdType.MESH)
            rx.start()
            rx.wait()
            cr.wait()

        _nsc_c = jax.named_scope("compute"); _nsc_c.__enter__()
        y = (psend[...].astype(jnp.float32)
             + precv[...].astype(jnp.float32)
             + rstage[...])
        rms = jnp.sqrt(jnp.mean(y * y, axis=-1, keepdims=True) + 1e-6)
        o = (y / rms * gamma_ref[...][None, :]).astype(jnp.bfloat16)
        gathA[pl.ds(c_me * R, R), :] = o[:, :HC]
        gathB[pl.ds((N_LINE * my_z + my_y) * R, R), :] = o[:, HC:]
        _nsc_c.__exit__(None, None, None)

        def z_peer(d):
            return (my_x, my_y, my_z + d)

        def y_peer(d):
            return (my_x, my_y + d, my_z)

        def unit_A_z(i):
            return gathA.at[pl.ds((N_LINE * my_y + i) * R, R)]

        def unit_B_y(i):
            return gathB.at[pl.ds((N_LINE * my_z + i) * R, R)]

        def unit_A_y(i):
            return gathA.at[pl.ds(i * N_LINE * R, N_LINE * R)]

        def unit_B_z(i):
            return gathB.at[pl.ds(i * N_LINE * R, N_LINE * R)]

        def step_send(phase, s, pos, unit, peer_fn):
            @pl.when((pos >= s) & (pos < N_LINE - 1))
            def _():
                r = pltpu.make_async_remote_copy(
                    src_ref=unit(pos - s), dst_ref=unit(pos - s),
                    send_sem=ag_send.at[phase, s, 0],
                    recv_sem=ag_recv.at[phase, s, 0],
                    device_id=peer_fn(1),
                    device_id_type=pl.DeviceIdType.MESH)
                r.start()

            @pl.when((pos > 0) & (pos + s <= N_LINE - 1))
            def _():
                r = pltpu.make_async_remote_copy(
                    src_ref=unit(pos + s), dst_ref=unit(pos + s),
                    send_sem=ag_send.at[phase, s, 1],
                    recv_sem=ag_recv.at[phase, s, 1],
                    device_id=peer_fn(-1),
                    device_id_type=pl.DeviceIdType.MESH)
                r.start()

        def step_wait(phase, s, pos, unit, peer_fn):
            @pl.when(pos >= s + 1)
            def _():
                r = pltpu.make_async_remote_copy(
                    src_ref=unit(pos - 1 - s), dst_ref=unit(pos - 1 - s),
                    send_sem=ag_send.at[phase, s, 0],
                    recv_sem=ag_recv.at[phase, s, 0],
                    device_id=peer_fn(-1),
                    device_id_type=pl.DeviceIdType.MESH)
                r.wait_recv()

            @pl.when(pos + 1 + s <= N_LINE - 1)
            def _():
                r = pltpu.make_async_remote_copy(
                    src_ref=unit(pos + 1 + s), dst_ref=unit(pos + 1 + s),
                    send_sem=ag_send.at[phase, s, 1],
                    recv_sem=ag_recv.at[phase, s, 1],
                    device_id=peer_fn(1),
                    device_id_type=pl.DeviceIdType.MESH)
                r.wait_recv()

            @pl.when((pos >= s) & (pos < N_LINE - 1))
            def _():
                r = pltpu.make_async_remote_copy(
                    src_ref=unit(pos - s), dst_ref=unit(pos - s),
                    send_sem=ag_send.at[phase, s, 0],
                    recv_sem=ag_recv.at[phase, s, 0],
                    device_id=peer_fn(1),
                    device_id_type=pl.DeviceIdType.MESH)
                r.wait_send()

            @pl.when((pos > 0) & (pos + s <= N_LINE - 1))
            def _():
                r = pltpu.make_async_remote_copy(
                    src_ref=unit(pos + s), dst_ref=unit(pos + s),
                    send_sem=ag_send.at[phase, s, 1],
                    recv_sem=ag_recv.at[phase, s, 1],
                    device_id=peer_fn(-1),
                    device_id_type=pl.DeviceIdType.MESH)
                r.wait_send()

        for s in range(N_LINE - 1):
            with jax.named_scope(f"agI#s={s}"):
                step_send(0, s, my_z, unit_A_z, z_peer)
                step_send(1, s, my_y, unit_B_y, y_peer)
                step_wait(0, s, my_z, unit_A_z, z_peer)
                step_wait(1, s, my_y, unit_B_y, y_peer)

        for s in range(N_LINE - 1):
            with jax.named_scope(f"agII#s={s}"):
                step_send(2, s, my_y, unit_A_y, y_peer)
                step_send(3, s, my_z, unit_B_z, z_peer)
                step_wait(2, s, my_y, unit_A_y, y_peer)
                step_wait(3, s, my_z, unit_B_z, z_peer)

        _nsc_s = jax.named_scope("store"); _nsc_s.__enter__()
        out_ref[:, 0:HC] = gathA[:, :]
        for yy in range(N_LINE):
            for zz in range(N_LINE):
                rows = pl.ds((N_LINE * yy + zz) * R, R)
                out_ref[rows, HC:D] = gathB[pl.ds((N_LINE * zz + yy) * R, R), :]
        _nsc_s.__exit__(None, None, None)

    return pl.pallas_call(
        body,
        out_shape=jax.ShapeDtypeStruct((M, D), jnp.bfloat16),
        in_specs=[
            pl.BlockSpec(memory_space=pltpu.MemorySpace.HBM),
            pl.BlockSpec(memory_space=pltpu.MemorySpace.HBM),
            pl.BlockSpec(memory_space=pltpu.MemorySpace.VMEM),
        ],
        out_specs=pl.BlockSpec(memory_space=pltpu.MemorySpace.VMEM),
        scratch_shapes=[
            pltpu.VMEM((M, HC), jnp.bfloat16),
            pltpu.VMEM((M, HC), jnp.bfloat16),
            pltpu.VMEM((R, D), jnp.float32),
            pltpu.VMEM((R, D), jnp.float32),
            pltpu.VMEM((R, D), jnp.bfloat16),
            pltpu.VMEM((R, D), jnp.bfloat16),
            pltpu.SemaphoreType.DMA,
            pltpu.SemaphoreType.DMA,
            pltpu.SemaphoreType.DMA,
            pltpu.SemaphoreType.DMA,
            pltpu.SemaphoreType.DMA((4, N_LINE - 1, 2)),
            pltpu.SemaphoreType.DMA((4, N_LINE - 1, 2)),
        ],
        compiler_params=pltpu.CompilerParams(collective_id=0),
    )(partial, resid, gamma)


# device time: 68283 ns/iter; 1.2136x vs baseline; 1.0430x over previous
import jax
import jax.numpy as jnp
from jax import lax
from jax.experimental import pallas as pl
from jax.experimental.pallas import tpu as pltpu

N = 4
R = 128


def kernel(partial, resid, gamma):
    _, M, D = partial.shape
    HC = D // 2

    def body(partial_ref, resid_ref, gamma_ref, out_ref,
             gathA, gathB, pstage, rstage, psend, precv,
             send_x, recv_x, dma_p, dma_r,
             ln_send, ln_recv, pc_send, pc_recv):
        my_x = lax.axis_index("x")
        my_y = lax.axis_index("y")
        my_z = lax.axis_index("z")
        xpeer = (1 - my_x, my_y, my_z)

        barrier_sem = pltpu.get_barrier_semaphore()
        pl.semaphore_signal(barrier_sem, inc=1, device_id=xpeer,
                            device_id_type=pl.DeviceIdType.MESH)

        @pl.when(my_y > 0)
        def _():
            pl.semaphore_signal(barrier_sem, inc=1,
                                device_id=(my_x, my_y - 1, my_z),
                                device_id_type=pl.DeviceIdType.MESH)

        @pl.when(my_y < N - 1)
        def _():
            pl.semaphore_signal(barrier_sem, inc=1,
                                device_id=(my_x, my_y + 1, my_z),
                                device_id_type=pl.DeviceIdType.MESH)

        @pl.when(my_z > 0)
        def _():
            pl.semaphore_signal(barrier_sem, inc=1,
                                device_id=(my_x, my_y, my_z - 1),
                                device_id_type=pl.DeviceIdType.MESH)

        @pl.when(my_z < N - 1)
        def _():
            pl.semaphore_signal(barrier_sem, inc=1,
                                device_id=(my_x, my_y, my_z + 1),
                                device_id_type=pl.DeviceIdType.MESH)

        c_me = N * my_y + my_z
        cp = pltpu.make_async_copy(
            partial_ref.at[0, pl.ds(c_me * R, R)], pstage, dma_p)
        cp.start()
        cr = pltpu.make_async_copy(
            resid_ref.at[pl.ds(c_me * R, R)], rstage, dma_r)
        cr.start()
        cp.wait()
        psend[...] = pstage[...].astype(jnp.bfloat16)

        n_nbrs = (1
                  + (my_y > 0).astype(jnp.int32)
                  + (my_y < N - 1).astype(jnp.int32)
                  + (my_z > 0).astype(jnp.int32)
                  + (my_z < N - 1).astype(jnp.int32))
        pl.semaphore_wait(barrier_sem, n_nbrs)

        rx = pltpu.make_async_remote_copy(
            src_ref=psend, dst_ref=precv, send_sem=send_x, recv_sem=recv_x,
            device_id=xpeer, device_id_type=pl.DeviceIdType.MESH)
        rx.start()
        rx.wait()
        cr.wait()

        y = (psend[...].astype(jnp.float32)
             + precv[...].astype(jnp.float32)
             + rstage[...])
        rms = jnp.sqrt(jnp.mean(y * y, axis=-1, keepdims=True) + 1e-6)
        o = (y / rms * gamma_ref[...][None, :]).astype(jnp.bfloat16)
        gathA[pl.ds(c_me * R, R), :] = o[:, :HC]
        gathB[pl.ds((N * my_z + my_y) * R, R), :] = o[:, HC:]

        A, B = 0, 1

        def peer(stream, axis_is_piece, d):
            if (stream == A) == (not axis_is_piece):
                return (my_x, my_y, my_z + d)
            return (my_x, my_y + d, my_z)

        def gref(stream):
            return gathA if stream == A else gathB

        def lpos(stream):
            return my_z if stream == A else my_y

        def ppos(stream):
            return my_y if stream == A else my_z

        def unit(stream, fp, lz):
            return gref(stream).at[pl.ds((N * fp + lz) * R, R)]

        def piece_send(stream, s, lz):
            pp = ppos(stream)
            @pl.when((pp - s >= 0) & (pp < N - 1))
            def _():
                r = pltpu.make_async_remote_copy(
                    src_ref=unit(stream, pp - s, lz),
                    dst_ref=unit(stream, pp - s, lz),
                    send_sem=pc_send.at[stream, 0, s, lz],
                    recv_sem=pc_recv.at[stream, 0, s, lz],
                    device_id=peer(stream, True, 1),
                    device_id_type=pl.DeviceIdType.MESH)
                r.start()

            @pl.when((pp + s <= N - 1) & (pp > 0))
            def _():
                r = pltpu.make_async_remote_copy(
                    src_ref=unit(stream, pp + s, lz),
                    dst_ref=unit(stream, pp + s, lz),
                    send_sem=pc_send.at[stream, 1, s, lz],
                    recv_sem=pc_recv.at[stream, 1, s, lz],
                    device_id=peer(stream, True, -1),
                    device_id_type=pl.DeviceIdType.MESH)
                r.start()

        def piece_wait_recv(stream, s, lz):
            pp = ppos(stream)
            @pl.when(pp >= s + 1)
            def _():
                r = pltpu.make_async_remote_copy(
                    src_ref=unit(stream, pp - 1 - s, lz),
                    dst_ref=unit(stream, pp - 1 - s, lz),
                    send_sem=pc_send.at[stream, 0, s, lz],
                    recv_sem=pc_recv.at[stream, 0, s, lz],
                    device_id=peer(stream, True, -1),
                    device_id_type=pl.DeviceIdType.MESH)
                r.wait_recv()

            @pl.when(pp + 1 + s <= N - 1)
            def _():
                r = pltpu.make_async_remote_copy(
                    src_ref=unit(stream, pp + 1 + s, lz),
                    dst_ref=unit(stream, pp + 1 + s, lz),
                    send_sem=pc_send.at[stream, 1, s, lz],
                    recv_sem=pc_recv.at[stream, 1, s, lz],
                    device_id=peer(stream, True, 1),
                    device_id_type=pl.DeviceIdType.MESH)
                r.wait_recv()

        def piece_wait_send(stream, s, lz):
            pp = ppos(stream)
            @pl.when((pp - s >= 0) & (pp < N - 1))
            def _():
                r = pltpu.make_async_remote_copy(
                    src_ref=unit(stream, pp - s, lz),
                    dst_ref=unit(stream, pp - s, lz),
                    send_sem=pc_send.at[stream, 0, s, lz],
                    recv_sem=pc_recv.at[stream, 0, s, lz],
                    device_id=peer(stream, True, 1),
                    device_id_type=pl.DeviceIdType.MESH)
                r.wait_send()

            @pl.when((pp + s <= N - 1) & (pp > 0))
            def _():
                r = pltpu.make_async_remote_copy(
                    src_ref=unit(stream, pp + s, lz),
                    dst_ref=unit(stream, pp + s, lz),
                    send_sem=pc_send.at[stream, 1, s, lz],
                    recv_sem=pc_recv.at[stream, 1, s, lz],
                    device_id=peer(stream, True, -1),
                    device_id_type=pl.DeviceIdType.MESH)
                r.wait_send()

        def line_descr(stream, s, dirn, idx):
            return pltpu.make_async_remote_copy(
                src_ref=unit(stream, ppos(stream), idx),
                dst_ref=unit(stream, ppos(stream), idx),
                send_sem=ln_send.at[stream, s, dirn],
                recv_sem=ln_recv.at[stream, s, dirn],
                device_id=peer(stream, False, 1 if dirn == 0 else -1),
                device_id_type=pl.DeviceIdType.MESH)

        def line_send(stream, s):
            lp = lpos(stream)
            @pl.when((lp >= s) & (lp < N - 1))
            def _():
                line_descr(stream, s, 0, lp - s).start()

            @pl.when((lp > 0) & (lp + s <= N - 1))
            def _():
                line_descr(stream, s, 1, lp + s).start()

        def line_wait_recv(stream, s):
            lp = lpos(stream)
            @pl.when(lp >= s + 1)
            def _():
                line_descr(stream, s, 0, lp - 1 - s).wait_recv()

            @pl.when(lp + 1 + s <= N - 1)
            def _():
                line_descr(stream, s, 1, lp + 1 + s).wait_recv()

        def line_wait_send(stream, s):
            lp = lpos(stream)
            @pl.when((lp >= s) & (lp < N - 1))
            def _():
                line_descr(stream, s, 0, lp - s).wait_send()

            @pl.when((lp > 0) & (lp + s <= N - 1))
            def _():
                line_descr(stream, s, 1, lp + s).wait_send()

        for st in (A, B):
            line_send(st, 0)
        for st in (A, B):
            piece_send(st, 0, lpos(st))

        for s in range(N - 1):
            if s > 0:
                for st in (A, B):
                    line_send(st, s)
            for st in (A, B):
                line_wait_recv(st, s)
            for st in (A, B):
                lp = lpos(st)
                @pl.when(lp - (s + 1) >= 0)
                def _(st=st, s=s, lp=lp):
                    piece_send(st, 0, lp - (s + 1))

                @pl.when(lp + (s + 1) <= N - 1)
                def _(st=st, s=s, lp=lp):
                    piece_send(st, 0, lp + (s + 1))

        for s in range(1, N - 1):
            for lz in range(N):
                for st in (A, B):
                    piece_wait_recv(st, s - 1, lz)
                for st in (A, B):
                    piece_send(st, s, lz)

        for lz in range(N):
            for st in (A, B):
                piece_wait_recv(st, N - 2, lz)

        for st in (A, B):
            for s in range(N - 1):
                line_wait_send(st, s)
        for st in (A, B):
            for s in range(N - 1):
                for lz in range(N):
                    piece_wait_send(st, s, lz)

        out_ref[:, 0:HC] = gathA[:, :]
        for yy in range(N):
            for zz in range(N):
                rows = pl.ds((N * yy + zz) * R, R)
                out_ref[rows, HC:D] = gathB[pl.ds((N * zz + yy) * R, R), :]

    return pl.pallas_call(
        body,
        out_shape=jax.ShapeDtypeStruct((M, D), jnp.bfloat16),
        in_specs=[
            pl.BlockSpec(memory_space=pltpu.MemorySpace.HBM),
            pl.BlockSpec(memory_space=pltpu.MemorySpace.HBM),
            pl.BlockSpec(memory_space=pltpu.MemorySpace.VMEM),
        ],
        out_specs=pl.BlockSpec(memory_space=pltpu.MemorySpace.VMEM),
        scratch_shapes=[
            pltpu.VMEM((M, HC), jnp.bfloat16),
            pltpu.VMEM((M, HC), jnp.bfloat16),
            pltpu.VMEM((R, D), jnp.float32),
            pltpu.VMEM((R, D), jnp.float32),
            pltpu.VMEM((R, D), jnp.bfloat16),
            pltpu.VMEM((R, D), jnp.bfloat16),
            pltpu.SemaphoreType.DMA,
            pltpu.SemaphoreType.DMA,
            pltpu.SemaphoreType.DMA,
            pltpu.SemaphoreType.DMA,
            pltpu.SemaphoreType.DMA((2, N - 1, 2)),
            pltpu.SemaphoreType.DMA((2, N - 1, 2)),
            pltpu.SemaphoreType.DMA((2, 2, N - 1, N)),
            pltpu.SemaphoreType.DMA((2, 2, N - 1, N)),
        ],
        compiler_params=pltpu.CompilerParams(collective_id=0),
    )(partial, resid, gamma)


# device time: 60157 ns/iter; 1.3775x vs baseline; 1.1351x over previous
import jax
import jax.numpy as jnp
from jax import lax
from jax.experimental import pallas as pl
from jax.experimental.pallas import tpu as pltpu

N = 4
R = 128
W = 768
XW = 512


def kernel(partial, resid, gamma):
    _, M, D = partial.shape

    def body(partial_ref, resid_ref, gamma_ref, out_ref,
             gathA, gathB, xrecv, pstage, rstage, psend, precv,
             send_x, recv_x, dma_p, dma_r,
             ln_send, ln_recv, pc_send, pc_recv,
             xl_send, xl_recv, xp_send, xp_recv):
        my_x = lax.axis_index("x")
        my_y = lax.axis_index("y")
        my_z = lax.axis_index("z")
        xpeer = (1 - my_x, my_y, my_z)

        barrier_sem = pltpu.get_barrier_semaphore()
        pl.semaphore_signal(barrier_sem, inc=1, device_id=xpeer,
                            device_id_type=pl.DeviceIdType.MESH)

        @pl.when(my_y > 0)
        def _():
            pl.semaphore_signal(barrier_sem, inc=1,
                                device_id=(my_x, my_y - 1, my_z),
                                device_id_type=pl.DeviceIdType.MESH)

        @pl.when(my_y < N - 1)
        def _():
            pl.semaphore_signal(barrier_sem, inc=1,
                                device_id=(my_x, my_y + 1, my_z),
                                device_id_type=pl.DeviceIdType.MESH)

        @pl.when(my_z > 0)
        def _():
            pl.semaphore_signal(barrier_sem, inc=1,
                                device_id=(my_x, my_y, my_z - 1),
                                device_id_type=pl.DeviceIdType.MESH)

        @pl.when(my_z < N - 1)
        def _():
            pl.semaphore_signal(barrier_sem, inc=1,
                                device_id=(my_x, my_y, my_z + 1),
                                device_id_type=pl.DeviceIdType.MESH)

        c_me = N * my_y + my_z
        cp = pltpu.make_async_copy(
            partial_ref.at[0, pl.ds(c_me * R, R)], pstage, dma_p)
        cp.start()
        cr = pltpu.make_async_copy(
            resid_ref.at[pl.ds(c_me * R, R)], rstage, dma_r)
        cr.start()
        cp.wait()
        psend[...] = pstage[...].astype(jnp.bfloat16)

        n_nbrs = (1
                  + (my_y > 0).astype(jnp.int32)
                  + (my_y < N - 1).astype(jnp.int32)
                  + (my_z > 0).astype(jnp.int32)
                  + (my_z < N - 1).astype(jnp.int32))
        pl.semaphore_wait(barrier_sem, n_nbrs)

        rx = pltpu.make_async_remote_copy(
            src_ref=psend, dst_ref=precv, send_sem=send_x, recv_sem=recv_x,
            device_id=xpeer, device_id_type=pl.DeviceIdType.MESH)
        rx.start()
        rx.wait()
        cr.wait()

        y = (psend[...].astype(jnp.float32)
             + precv[...].astype(jnp.float32)
             + rstage[...])
        rms = jnp.sqrt(jnp.mean(y * y, axis=-1, keepdims=True) + 1e-6)
        o = (y / rms * gamma_ref[...][None, :]).astype(jnp.bfloat16)
        @pl.when(my_x == 0)
        def _():
            gathA[pl.ds(c_me * R, R), :] = o[:, 0:W]
            gathB[pl.ds((N * my_z + my_y) * R, R), :] = o[:, W:2 * W]
            xrecv[pl.ds(c_me * R, R), :] = o[:, 3 * XW:D]

        @pl.when(my_x == 1)
        def _():
            gathA[pl.ds(c_me * R, R), :] = o[:, XW:XW + W]
            gathB[pl.ds((N * my_z + my_y) * R, R), :] = o[:, XW + W:D]
            xrecv[pl.ds(c_me * R, R), :] = o[:, 0:XW]

        A, B = 0, 1

        def peer(stream, axis_is_piece, d):
            if (stream == A) == (not axis_is_piece):
                return (my_x, my_y, my_z + d)
            return (my_x, my_y + d, my_z)

        def gref(stream):
            return gathA if stream == A else gathB

        def lpos(stream):
            return my_z if stream == A else my_y

        def ppos(stream):
            return my_y if stream == A else my_z

        def unit(stream, fp, lz):
            return gref(stream).at[pl.ds((N * fp + lz) * R, R)]

        def xfwd(stream, fp, lz, sem_s, sem_r):
            xok = (my_x == 0) if stream == A else (my_x == 1)
            @pl.when(xok)
            def _():
                if stream == A:
                    src = gathA.at[pl.ds((N * fp + lz) * R, R), pl.ds(0, XW)]
                    dst_row = (N * fp + lz) * R
                else:
                    src = gathB.at[pl.ds((N * fp + lz) * R, R),
                                   pl.ds(W - XW, XW)]
                    dst_row = (N * lz + fp) * R
                r = pltpu.make_async_remote_copy(
                    src_ref=src,
                    dst_ref=xrecv.at[pl.ds(dst_row, R)],
                    send_sem=sem_s, recv_sem=sem_r,
                    device_id=xpeer, device_id_type=pl.DeviceIdType.MESH)
                r.start()

        def piece_send(stream, s, lz):
            pp = ppos(stream)
            @pl.when((pp - s >= 0) & (pp < N - 1))
            def _():
                r = pltpu.make_async_remote_copy(
                    src_ref=unit(stream, pp - s, lz),
                    dst_ref=unit(stream, pp - s, lz),
                    send_sem=pc_send.at[stream, 0, s, lz],
                    recv_sem=pc_recv.at[stream, 0, s, lz],
                    device_id=peer(stream, True, 1),
                    device_id_type=pl.DeviceIdType.MESH)
                r.start()

            @pl.when((pp + s <= N - 1) & (pp > 0))
            def _():
                r = pltpu.make_async_remote_copy(
                    src_ref=unit(stream, pp + s, lz),
                    dst_ref=unit(stream, pp + s, lz),
                    send_sem=pc_send.at[stream, 1, s, lz],
                    recv_sem=pc_recv.at[stream, 1, s, lz],
                    device_id=peer(stream, True, -1),
                    device_id_type=pl.DeviceIdType.MESH)
                r.start()

        def piece_wait_recv(stream, h, lz):
            pp = ppos(stream)
            @pl.when(pp >= h + 1)
            def _():
                r = pltpu.make_async_remote_copy(
                    src_ref=unit(stream, pp - 1 - h, lz),
                    dst_ref=unit(stream, pp - 1 - h, lz),
                    send_sem=pc_send.at[stream, 0, h, lz],
                    recv_sem=pc_recv.at[stream, 0, h, lz],
                    device_id=peer(stream, True, -1),
                    device_id_type=pl.DeviceIdType.MESH)
                r.wait_recv()
                xfwd(stream, pp - 1 - h, lz,
                     xp_send.at[h, lz, 0], xp_recv.at[h, lz, 0])

            @pl.when(pp + 1 + h <= N - 1)
            def _():
                r = pltpu.make_async_remote_copy(
                    src_ref=unit(stream, pp + 1 + h, lz),
                    dst_ref=unit(stream, pp + 1 + h, lz),
                    send_sem=pc_send.at[stream, 1, h, lz],
                    recv_sem=pc_recv.at[stream, 1, h, lz],
                    device_id=peer(stream, True, 1),
                    device_id_type=pl.DeviceIdType.MESH)
                r.wait_recv()
                xfwd(stream, pp + 1 + h, lz,
                     xp_send.at[h, lz, 1], xp_recv.at[h, lz, 1])

        def piece_wait_send(stream, s, lz):
            pp = ppos(stream)
            @pl.when((pp - s >= 0) & (pp < N - 1))
            def _():
                r = pltpu.make_async_remote_copy(
                    src_ref=unit(stream, pp - s, lz),
                    dst_ref=unit(stream, pp - s, lz),
                    send_sem=pc_send.at[stream, 0, s, lz],
                    recv_sem=pc_recv.at[stream, 0, s, lz],
                    device_id=peer(stream, True, 1),
                    device_id_type=pl.DeviceIdType.MESH)
                r.wait_send()

            @pl.when((pp + s <= N - 1) & (pp > 0))
            def _():
                r = pltpu.make_async_remote_copy(
                    src_ref=unit(stream, pp + s, lz),
                    dst_ref=unit(stream, pp + s, lz),
                    send_sem=pc_send.at[stream, 1, s, lz],
                    recv_sem=pc_recv.at[stream, 1, s, lz],
                    device_id=peer(stream, True, -1),
                    device_id_type=pl.DeviceIdType.MESH)
                r.wait_send()

        def line_descr(stream, s, dirn, idx):
            return pltpu.make_async_remote_copy(
                src_ref=unit(stream, ppos(stream), idx),
                dst_ref=unit(stream, ppos(stream), idx),
                send_sem=ln_send.at[stream, s, dirn],
                recv_sem=ln_recv.at[stream, s, dirn],
                device_id=peer(stream, False, 1 if dirn == 0 else -1),
                device_id_type=pl.DeviceIdType.MESH)

        def line_send(stream, s):
            lp = lpos(stream)
            @pl.when((lp >= s) & (lp < N - 1))
            def _():
                line_descr(stream, s, 0, lp - s).start()

            @pl.when((lp > 0) & (lp + s <= N - 1))
            def _():
                line_descr(stream, s, 1, lp + s).start()

        def line_wait_recv(stream, s):
            lp = lpos(stream)
            @pl.when(lp >= s + 1)
            def _():
                line_descr(stream, s, 0, lp - 1 - s).wait_recv()
                xfwd(stream, ppos(stream), lp - 1 - s,
                     xl_send.at[s, 0], xl_recv.at[s, 0])

            @pl.when(lp + 1 + s <= N - 1)
            def _():
                line_descr(stream, s, 1, lp + 1 + s).wait_recv()
                xfwd(stream, ppos(stream), lp + 1 + s,
                     xl_send.at[s, 1], xl_recv.at[s, 1])

        def line_wait_send(stream, s):
            lp = lpos(stream)
            @pl.when((lp >= s) & (lp < N - 1))
            def _():
                line_descr(stream, s, 0, lp - s).wait_send()

            @pl.when((lp > 0) & (lp + s <= N - 1))
            def _():
                line_descr(stream, s, 1, lp + s).wait_send()

        for st in (A, B):
            line_send(st, 0)
        for st in (A, B):
            piece_send(st, 0, lpos(st))

        for s in range(N - 1):
            if s > 0:
                for st in (A, B):
                    line_send(st, s)
            for st in (A, B):
                line_wait_recv(st, s)
            for st in (A, B):
                lp = lpos(st)
                @pl.when(lp - (s + 1) >= 0)
                def _(st=st, s=s, lp=lp):
                    piece_send(st, 0, lp - (s + 1))

                @pl.when(lp + (s + 1) <= N - 1)
                def _(st=st, s=s, lp=lp):
                    piece_send(st, 0, lp + (s + 1))

        for s in range(1, N - 1):
            for lz in range(N):
                for st in (A, B):
                    piece_wait_recv(st, s - 1, lz)
                for st in (A, B):
                    piece_send(st, s, lz)

        for lz in range(N):
            for st in (A, B):
                piece_wait_recv(st, N - 2, lz)

        plp = jnp.where(my_x == 0, my_y, my_z)
        ppp = jnp.where(my_x == 0, my_z, my_y)

        def xrow(fp, lz):
            return jnp.where(my_x == 0, N * lz + fp, N * fp + lz) * R

        def xwait(fp, lz, sem_s, sem_r):
            r = pltpu.make_async_remote_copy(
                src_ref=xrecv.at[pl.ds(xrow(fp, lz), R)],
                dst_ref=xrecv.at[pl.ds(xrow(fp, lz), R)],
                send_sem=sem_s, recv_sem=sem_r,
                device_id=xpeer, device_id_type=pl.DeviceIdType.MESH)
            r.wait_recv()

        for s in range(N - 1):
            @pl.when(plp >= s + 1)
            def _(s=s):
                xwait(ppp, plp - 1 - s, xl_send.at[s, 0], xl_recv.at[s, 0])

            @pl.when(plp + 1 + s <= N - 1)
            def _(s=s):
                xwait(ppp, plp + 1 + s, xl_send.at[s, 1], xl_recv.at[s, 1])

        for h in range(N - 1):
            for lz in range(N):
                @pl.when(ppp >= h + 1)
                def _(h=h, lz=lz):
                    xwait(ppp - 1 - h, lz,
                          xp_send.at[h, lz, 0], xp_recv.at[h, lz, 0])

                @pl.when(ppp + 1 + h <= N - 1)
                def _(h=h, lz=lz):
                    xwait(ppp + 1 + h, lz,
                          xp_send.at[h, lz, 1], xp_recv.at[h, lz, 1])

        for st in (A, B):
            for s in range(N - 1):
                line_wait_send(st, s)
        for st in (A, B):
            for s in range(N - 1):
                for lz in range(N):
                    piece_wait_send(st, s, lz)

        def xdrain(sem_s, sem_r, cond):
            @pl.when(cond)
            def _():
                r = pltpu.make_async_remote_copy(
                    src_ref=xrecv.at[pl.ds(0, R)],
                    dst_ref=xrecv.at[pl.ds(0, R)],
                    send_sem=sem_s, recv_sem=sem_r,
                    device_id=xpeer, device_id_type=pl.DeviceIdType.MESH)
                r.wait_send()

        mlp = jnp.where(my_x == 0, my_z, my_y)
        mpp = jnp.where(my_x == 0, my_y, my_z)
        for s in range(N - 1):
            xdrain(xl_send.at[s, 0], xl_recv.at[s, 0], mlp >= s + 1)
            xdrain(xl_send.at[s, 1], xl_recv.at[s, 1], mlp + 1 + s <= N - 1)
        for h in range(N - 1):
            for lz in range(N):
                xdrain(xp_send.at[h, lz, 0], xp_recv.at[h, lz, 0],
                       mpp >= h + 1)
                xdrain(xp_send.at[h, lz, 1], xp_recv.at[h, lz, 1],
                       mpp + 1 + h <= N - 1)

        @pl.when(my_x == 0)
        def _():
            out_ref[:, 0:W] = gathA[:, :]
            for yy in range(N):
                for zz in range(N):
                    rows = pl.ds((N * yy + zz) * R, R)
                    out_ref[rows, W:2 * W] = (
                        gathB[pl.ds((N * zz + yy) * R, R), :])
            out_ref[:, 2 * W:D] = xrecv[:, :]

        @pl.when(my_x == 1)
        def _():
            out_ref[:, XW:XW + W] = gathA[:, :]
            for yy in range(N):
                for zz in range(N):
                    rows = pl.ds((N * yy + zz) * R, R)
                    out_ref[rows, XW + W:D] = (
                        gathB[pl.ds((N * zz + yy) * R, R), :])
            out_ref[:, 0:XW] = xrecv[:, :]

    return pl.pallas_call(
        body,
        out_shape=jax.ShapeDtypeStruct((M, D), jnp.bfloat16),
        in_specs=[
            pl.BlockSpec(memory_space=pltpu.MemorySpace.HBM),
            pl.BlockSpec(memory_space=pltpu.MemorySpace.HBM),
            pl.BlockSpec(memory_space=pltpu.MemorySpace.VMEM),
        ],
        out_specs=pl.BlockSpec(memory_space=pltpu.MemorySpace.VMEM),
        scratch_shapes=[
            pltpu.VMEM((M, W), jnp.bfloat16),
            pltpu.VMEM((M, W), jnp.bfloat16),
            pltpu.VMEM((M, XW), jnp.bfloat16),
            pltpu.VMEM((R, D), jnp.float32),
            pltpu.VMEM((R, D), jnp.float32),
            pltpu.VMEM((R, D), jnp.bfloat16),
            pltpu.VMEM((R, D), jnp.bfloat16),
            pltpu.SemaphoreType.DMA,
            pltpu.SemaphoreType.DMA,
            pltpu.SemaphoreType.DMA,
            pltpu.SemaphoreType.DMA,
            pltpu.SemaphoreType.DMA((2, N - 1, 2)),
            pltpu.SemaphoreType.DMA((2, N - 1, 2)),
            pltpu.SemaphoreType.DMA((2, 2, N - 1, N)),
            pltpu.SemaphoreType.DMA((2, 2, N - 1, N)),
            pltpu.SemaphoreType.DMA((N - 1, 2)),
            pltpu.SemaphoreType.DMA((N - 1, 2)),
            pltpu.SemaphoreType.DMA((N - 1, N, 2)),
            pltpu.SemaphoreType.DMA((N - 1, N, 2)),
        ],
        compiler_params=pltpu.CompilerParams(collective_id=0),
    )(partial, resid, gamma)


# device time: 59592 ns/iter; 1.3906x vs baseline; 1.0095x over previous
import jax
import jax.numpy as jnp
from jax import lax
from jax.experimental import pallas as pl
from jax.experimental.pallas import tpu as pltpu

N = 4
R = 128
W = 768
XW = 512


def kernel(partial, resid, gamma):
    _, M, D = partial.shape

    def body(partial_ref, resid_ref, gamma_ref, out_ref,
             pstage, rstage, psend, precv,
             send_x, recv_x, dma_p, dma_r,
             ln_send, ln_recv, pc_send, pc_recv,
             xl_send, xl_recv, xp_send, xp_recv):
        my_x = lax.axis_index("x")
        my_y = lax.axis_index("y")
        my_z = lax.axis_index("z")
        xpeer = (1 - my_x, my_y, my_z)

        barrier_sem = pltpu.get_barrier_semaphore()
        pl.semaphore_signal(barrier_sem, inc=1, device_id=xpeer,
                            device_id_type=pl.DeviceIdType.MESH)

        @pl.when(my_y > 0)
        def _():
            pl.semaphore_signal(barrier_sem, inc=1,
                                device_id=(my_x, my_y - 1, my_z),
                                device_id_type=pl.DeviceIdType.MESH)

        @pl.when(my_y < N - 1)
        def _():
            pl.semaphore_signal(barrier_sem, inc=1,
                                device_id=(my_x, my_y + 1, my_z),
                                device_id_type=pl.DeviceIdType.MESH)

        @pl.when(my_z > 0)
        def _():
            pl.semaphore_signal(barrier_sem, inc=1,
                                device_id=(my_x, my_y, my_z - 1),
                                device_id_type=pl.DeviceIdType.MESH)

        @pl.when(my_z < N - 1)
        def _():
            pl.semaphore_signal(barrier_sem, inc=1,
                                device_id=(my_x, my_y, my_z + 1),
                                device_id_type=pl.DeviceIdType.MESH)

        c_me = N * my_y + my_z
        cp = pltpu.make_async_copy(
            partial_ref.at[0, pl.ds(c_me * R, R)], pstage, dma_p)
        cp.start()
        cr = pltpu.make_async_copy(
            resid_ref.at[pl.ds(c_me * R, R)], rstage, dma_r)
        cr.start()
        cp.wait()
        psend[...] = pstage[...].astype(jnp.bfloat16)

        n_nbrs = (1
                  + (my_y > 0).astype(jnp.int32)
                  + (my_y < N - 1).astype(jnp.int32)
                  + (my_z > 0).astype(jnp.int32)
                  + (my_z < N - 1).astype(jnp.int32))
        pl.semaphore_wait(barrier_sem, n_nbrs)

        rx = pltpu.make_async_remote_copy(
            src_ref=psend, dst_ref=precv, send_sem=send_x, recv_sem=recv_x,
            device_id=xpeer, device_id_type=pl.DeviceIdType.MESH)
        rx.start()
        rx.wait()
        cr.wait()

        y = (psend[...].astype(jnp.float32)
             + precv[...].astype(jnp.float32)
             + rstage[...])
        rms = jnp.sqrt(jnp.mean(y * y, axis=-1, keepdims=True) + 1e-6)
        o = (y / rms * gamma_ref[...][None, :]).astype(jnp.bfloat16)
        out_ref[pl.ds(c_me * R, R), :] = o

        A, B = 0, 1

        def peer(stream, axis_is_piece, d):
            if (stream == A) == (not axis_is_piece):
                return (my_x, my_y, my_z + d)
            return (my_x, my_y + d, my_z)

        def lpos(stream):
            return my_z if stream == A else my_y

        def ppos(stream):
            return my_y if stream == A else my_z

        def unit(stream, fp, lz):
            if stream == A:
                return out_ref.at[pl.ds((N * fp + lz) * R, R),
                                  pl.ds(my_x * XW, W)]
            return out_ref.at[pl.ds((N * lz + fp) * R, R),
                              pl.ds(my_x * XW + W, W)]

        def xfwd(stream, fp, lz, sem_s, sem_r):
            xok = (my_x == 0) if stream == A else (my_x == 1)
            @pl.when(xok)
            def _():
                if stream == A:
                    blk = out_ref.at[pl.ds((N * fp + lz) * R, R),
                                     pl.ds(0, XW)]
                else:
                    blk = out_ref.at[pl.ds((N * lz + fp) * R, R),
                                     pl.ds(3 * XW, XW)]
                r = pltpu.make_async_remote_copy(
                    src_ref=blk, dst_ref=blk,
                    send_sem=sem_s, recv_sem=sem_r,
                    device_id=xpeer, device_id_type=pl.DeviceIdType.MESH)
                r.start()

        def piece_send(stream, s, lz):
            pp = ppos(stream)
            @pl.when((pp - s >= 0) & (pp < N - 1))
            def _():
                r = pltpu.make_async_remote_copy(
                    src_ref=unit(stream, pp - s, lz),
                    dst_ref=unit(stream, pp - s, lz),
                    send_sem=pc_send.at[stream, 0, s, lz],
                    recv_sem=pc_recv.at[stream, 0, s, lz],
                    device_id=peer(stream, True, 1),
                    device_id_type=pl.DeviceIdType.MESH)
                r.start()

            @pl.when((pp + s <= N - 1) & (pp > 0))
            def _():
                r = pltpu.make_async_remote_copy(
                    src_ref=unit(stream, pp + s, lz),
                    dst_ref=unit(stream, pp + s, lz),
                    send_sem=pc_send.at[stream, 1, s, lz],
                    recv_sem=pc_recv.at[stream, 1, s, lz],
                    device_id=peer(stream, True, -1),
                    device_id_type=pl.DeviceIdType.MESH)
                r.start()

        def piece_wait_recv(stream, h, lz):
            pp = ppos(stream)
            @pl.when(pp >= h + 1)
            def _():
                r = pltpu.make_async_remote_copy(
                    src_ref=unit(stream, pp - 1 - h, lz),
                    dst_ref=unit(stream, pp - 1 - h, lz),
                    send_sem=pc_send.at[stream, 0, h, lz],
                    recv_sem=pc_recv.at[stream, 0, h, lz],
                    device_id=peer(stream, True, -1),
                    device_id_type=pl.DeviceIdType.MESH)
                r.wait_recv()
                xfwd(stream, pp - 1 - h, lz,
                     xp_send.at[h, lz, 0], xp_recv.at[h, lz, 0])

            @pl.when(pp + 1 + h <= N - 1)
            def _():
                r = pltpu.make_async_remote_copy(
                    src_ref=unit(stream, pp + 1 + h, lz),
                    dst_ref=unit(stream, pp + 1 + h, lz),
                    send_sem=pc_send.at[stream, 1, h, lz],
                    recv_sem=pc_recv.at[stream, 1, h, lz],
                    device_id=peer(stream, True, 1),
                    device_id_type=pl.DeviceIdType.MESH)
                r.wait_recv()
                xfwd(stream, pp + 1 + h, lz,
                     xp_send.at[h, lz, 1], xp_recv.at[h, lz, 1])

        def piece_wait_send(stream, s, lz):
            pp = ppos(stream)
            @pl.when((pp - s >= 0) & (pp < N - 1))
            def _():
                r = pltpu.make_async_remote_copy(
                    src_ref=unit(stream, pp - s, lz),
                    dst_ref=unit(stream, pp - s, lz),
                    send_sem=pc_send.at[stream, 0, s, lz],
                    recv_sem=pc_recv.at[stream, 0, s, lz],
                    device_id=peer(stream, True, 1),
                    device_id_type=pl.DeviceIdType.MESH)
                r.wait_send()

            @pl.when((pp + s <= N - 1) & (pp > 0))
            def _():
                r = pltpu.make_async_remote_copy(
                    src_ref=unit(stream, pp + s, lz),
                    dst_ref=unit(stream, pp + s, lz),
                    send_sem=pc_send.at[stream, 1, s, lz],
                    recv_sem=pc_recv.at[stream, 1, s, lz],
                    device_id=peer(stream, True, -1),
                    device_id_type=pl.DeviceIdType.MESH)
                r.wait_send()

        def line_descr(stream, s, dirn, idx):
            return pltpu.make_async_remote_copy(
                src_ref=unit(stream, ppos(stream), idx),
                dst_ref=unit(stream, ppos(stream), idx),
                send_sem=ln_send.at[stream, s, dirn],
                recv_sem=ln_recv.at[stream, s, dirn],
                device_id=peer(stream, False, 1 if dirn == 0 else -1),
                device_id_type=pl.DeviceIdType.MESH)

        def line_send(stream, s):
            lp = lpos(stream)
            @pl.when((lp >= s) & (lp < N - 1))
            def _():
                line_descr(stream, s, 0, lp - s).start()

            @pl.when((lp > 0) & (lp + s <= N - 1))
            def _():
                line_descr(stream, s, 1, lp + s).start()

        def line_wait_recv(stream, s):
            lp = lpos(stream)
            @pl.when(lp >= s + 1)
            def _():
                line_descr(stream, s, 0, lp - 1 - s).wait_recv()
                xfwd(stream, ppos(stream), lp - 1 - s,
                     xl_send.at[s, 0], xl_recv.at[s, 0])

            @pl.when(lp + 1 + s <= N - 1)
            def _():
                line_descr(stream, s, 1, lp + 1 + s).wait_recv()
                xfwd(stream, ppos(stream), lp + 1 + s,
                     xl_send.at[s, 1], xl_recv.at[s, 1])

        def line_wait_send(stream, s):
            lp = lpos(stream)
            @pl.when((lp >= s) & (lp < N - 1))
            def _():
                line_descr(stream, s, 0, lp - s).wait_send()

            @pl.when((lp > 0) & (lp + s <= N - 1))
            def _():
                line_descr(stream, s, 1, lp + s).wait_send()

        for st in (A, B):
            line_send(st, 0)
        for st in (A, B):
            piece_send(st, 0, lpos(st))

        for s in range(N - 1):
            if s > 0:
                for st in (A, B):
                    line_send(st, s)
            for st in (A, B):
                line_wait_recv(st, s)
            for st in (A, B):
                lp = lpos(st)
                @pl.when(lp - (s + 1) >= 0)
                def _(st=st, s=s, lp=lp):
                    piece_send(st, 0, lp - (s + 1))

                @pl.when(lp + (s + 1) <= N - 1)
                def _(st=st, s=s, lp=lp):
                    piece_send(st, 0, lp + (s + 1))

        for s in range(1, N - 1):
            for lz in range(N):
                for st in (A, B):
                    piece_wait_recv(st, s - 1, lz)
                for st in (A, B):
                    piece_send(st, s, lz)

        for lz in range(N):
            for st in (A, B):
                piece_wait_recv(st, N - 2, lz)

        plp = jnp.where(my_x == 0, my_y, my_z)
        ppp = jnp.where(my_x == 0, my_z, my_y)
        xcol = (1 - my_x) * 3 * XW

        def xrow(fp, lz):
            return jnp.where(my_x == 0, N * lz + fp, N * fp + lz) * R

        def xwait(fp, lz, sem_s, sem_r):
            blk = out_ref.at[pl.ds(xrow(fp, lz), R), pl.ds(xcol, XW)]
            r = pltpu.make_async_remote_copy(
                src_ref=blk, dst_ref=blk, send_sem=sem_s, recv_sem=sem_r,
                device_id=xpeer, device_id_type=pl.DeviceIdType.MESH)
            r.wait_recv()

        for s in range(N - 1):
            @pl.when(plp >= s + 1)
            def _(s=s):
                xwait(ppp, plp - 1 - s, xl_send.at[s, 0], xl_recv.at[s, 0])

            @pl.when(plp + 1 + s <= N - 1)
            def _(s=s):
                xwait(ppp, plp + 1 + s, xl_send.at[s, 1], xl_recv.at[s, 1])

        for h in range(N - 1):
            for lz in range(N):
                @pl.when(ppp >= h + 1)
                def _(h=h, lz=lz):
                    xwait(ppp - 1 - h, lz,
                          xp_send.at[h, lz, 0], xp_recv.at[h, lz, 0])

                @pl.when(ppp + 1 + h <= N - 1)
                def _(h=h, lz=lz):
                    xwait(ppp + 1 + h, lz,
                          xp_send.at[h, lz, 1], xp_recv.at[h, lz, 1])

        for st in (A, B):
            for s in range(N - 1):
                line_wait_send(st, s)
        for st in (A, B):
            for s in range(N - 1):
                for lz in range(N):
                    piece_wait_send(st, s, lz)

        def xdrain(sem_s, sem_r, cond):
            @pl.when(cond)
            def _():
                blk = out_ref.at[pl.ds(0, R), pl.ds(0, XW)]
                r = pltpu.make_async_remote_copy(
                    src_ref=blk, dst_ref=blk, send_sem=sem_s, recv_sem=sem_r,
                    device_id=xpeer, device_id_type=pl.DeviceIdType.MESH)
                r.wait_send()

        mlp = jnp.where(my_x == 0, my_z, my_y)
        mpp = jnp.where(my_x == 0, my_y, my_z)
        for s in range(N - 1):
            xdrain(xl_send.at[s, 0], xl_recv.at[s, 0], mlp >= s + 1)
            xdrain(xl_send.at[s, 1], xl_recv.at[s, 1], mlp + 1 + s <= N - 1)
        for h in range(N - 1):
            for lz in range(N):
                xdrain(xp_send.at[h, lz, 0], xp_recv.at[h, lz, 0],
                       mpp >= h + 1)
                xdrain(xp_send.at[h, lz, 1], xp_recv.at[h, lz, 1],
                       mpp + 1 + h <= N - 1)

    return pl.pallas_call(
        body,
        out_shape=jax.ShapeDtypeStruct((M, D), jnp.bfloat16),
        in_specs=[
            pl.BlockSpec(memory_space=pltpu.MemorySpace.HBM),
            pl.BlockSpec(memory_space=pltpu.MemorySpace.HBM),
            pl.BlockSpec(memory_space=pltpu.MemorySpace.VMEM),
        ],
        out_specs=pl.BlockSpec(memory_space=pltpu.MemorySpace.VMEM),
        scratch_shapes=[
            pltpu.VMEM((R, D), jnp.float32),
            pltpu.VMEM((R, D), jnp.float32),
            pltpu.VMEM((R, D), jnp.bfloat16),
            pltpu.VMEM((R, D), jnp.bfloat16),
            pltpu.SemaphoreType.DMA,
            pltpu.SemaphoreType.DMA,
            pltpu.SemaphoreType.DMA,
            pltpu.SemaphoreType.DMA,
            pltpu.SemaphoreType.DMA((2, N - 1, 2)),
            pltpu.SemaphoreType.DMA((2, N - 1, 2)),
            pltpu.SemaphoreType.DMA((2, 2, N - 1, N)),
            pltpu.SemaphoreType.DMA((2, 2, N - 1, N)),
            pltpu.SemaphoreType.DMA((N - 1, 2)),
            pltpu.SemaphoreType.DMA((N - 1, 2)),
            pltpu.SemaphoreType.DMA((N - 1, N, 2)),
            pltpu.SemaphoreType.DMA((N - 1, N, 2)),
        ],
        compiler_params=pltpu.CompilerParams(collective_id=0),
    )(partial, resid, gamma)
